# baseline (speedup 1.0000x reference)
import functools

import jax
import jax.numpy as jnp
from jax import lax
from jax.experimental import pallas as pl
from jax.experimental.pallas import tpu as pltpu
from jax.experimental.pallas import tpu_sc as plsc

_EMBED_DIM = 32
_NUM_IDS = 16
_LANES = 16

@functools.cache
def _sc_pool_kernel():
    mesh = plsc.VectorSubcoreMesh(core_axis_name="c", subcore_axis_name="s")

    @functools.partial(
        pl.kernel,
        out_type=jax.ShapeDtypeStruct((_EMBED_DIM,), jnp.float32),
        mesh=mesh,
        compiler_params=pltpu.CompilerParams(use_tc_tiling_on_sc=False),
        scratch_types=[
            pltpu.VMEM((_NUM_IDS,), jnp.int32),
            pltpu.VMEM((_NUM_IDS, _EMBED_DIM), jnp.float32),
            pltpu.VMEM((_EMBED_DIM,), jnp.float32),
            pltpu.SemaphoreType.DMA,
        ],
    )
    def _sc_pool(table_hbm, idx_hbm, out_hbm, idx_v, rows_v, pooled_v, sem):
        wid = lax.axis_index("s") * 2 + lax.axis_index("c")

        @pl.when(wid == 0)
        def _():
            pltpu.sync_copy(idx_hbm, idx_v)
            pltpu.async_copy(table_hbm.at[idx_v], rows_v, sem).wait()
            ids = idx_v[...]
            mask = jnp.where(ids > 0, jnp.float32(1.0), jnp.float32(0.0))
            cnt = jnp.float32(0.0)
            for i in range(5):
                cnt = cnt + mask[i]
            inv = jnp.float32(1.0)
            for n in (2, 3, 4, 5):
                inv = jnp.where(cnt > (n - 0.5), jnp.float32(1.0 / n), inv)
            for c in range(_EMBED_DIM // _LANES):
                acc = rows_v[0, pl.ds(c * _LANES, _LANES)]
                for i in range(1, _NUM_IDS):
                    acc = acc + rows_v[i, pl.ds(c * _LANES, _LANES)]
                pooled_v[pl.ds(c * _LANES, _LANES)] = acc * inv
            pltpu.sync_copy(pooled_v, out_hbm)

    return _sc_pool


def _tc_mlp_body(feats_ref, w1_ref, b1_ref, w2_ref, b2_ref, w3_ref, b3_ref,
                 out_ref):
    h = jnp.dot(feats_ref[...], w1_ref[...],
                preferred_element_type=jnp.float32)
    h = jnp.maximum(h + b1_ref[...], 0.0)
    h = jnp.dot(h, w2_ref[...], preferred_element_type=jnp.float32)
    h = jnp.maximum(h + b2_ref[...], 0.0)
    out_ref[...] = (jnp.dot(h, w3_ref[...],
                            preferred_element_type=jnp.float32) + b3_ref[...])


def _tc_mlp(feats, W1, b1, W2, b2, W3, b3):
    return pl.pallas_call(
        _tc_mlp_body,
        out_shape=jax.ShapeDtypeStruct((1, W3.shape[1]), jnp.float32),
    )(feats, W1, b1, W2, b2, W3, b3)


def kernel(scalars, selection_mask, hand, hand_type, deck, jokers, emb_table,
           W1, b1, W2, b2, W3, b3):
    ids = jokers[:, 0].astype(jnp.int32)
    idx16 = jnp.zeros((_NUM_IDS,), jnp.int32).at[: ids.shape[0]].set(ids)
    pooled = _sc_pool_kernel()(emb_table, idx16)
    feats = jnp.concatenate([
        scalars, selection_mask, hand.reshape(-1), hand_type, deck, pooled,
        jokers[:, 1],
    ]).reshape(1, -1)
    out = _tc_mlp(feats, W1, b1.reshape(1, -1), W2, b2.reshape(1, -1), W3,
                  b3.reshape(1, -1))
    return out.reshape(-1)


# speedup vs baseline: 1.6497x; 1.6497x over previous
import functools

import jax
import jax.numpy as jnp
from jax import lax
from jax.experimental import pallas as pl
from jax.experimental.pallas import tpu as pltpu
from jax.experimental.pallas import tpu_sc as plsc

_EMBED_DIM = 32
_NUM_IDS = 16
_NUM_SLOTS = 5
_LANES = 16

@functools.cache
def _sc_pool_kernel():
    mesh = plsc.VectorSubcoreMesh(core_axis_name="c", subcore_axis_name="s")

    @functools.partial(
        pl.kernel,
        out_type=jax.ShapeDtypeStruct((_EMBED_DIM,), jnp.float32),
        mesh=mesh,
        scratch_types=[
            pltpu.VMEM((_NUM_IDS,), jnp.int32),
            pltpu.VMEM((_NUM_SLOTS, _EMBED_DIM), jnp.float32),
            pltpu.VMEM((_EMBED_DIM,), jnp.float32),
            pltpu.SemaphoreType.DMA,
        ],
    )
    def _sc_pool(table_hbm, idx_hbm, out_hbm, idx_v, rows_v, pooled_v, sem):
        wid = lax.axis_index("s") * 2 + lax.axis_index("c")

        @pl.when(wid == 0)
        def _():
            pltpu.sync_copy(idx_hbm, idx_v)
            ids = idx_v[...]
            copies = [
                pltpu.async_copy(
                    table_hbm.at[pl.ds(ids[i], 1), :],
                    rows_v.at[pl.ds(i, 1), :],
                    sem,
                )
                for i in range(_NUM_SLOTS)
            ]
            mask = jnp.where(ids > 0, jnp.float32(1.0), jnp.float32(0.0))
            cnt = jnp.float32(0.0)
            for i in range(_NUM_SLOTS):
                cnt = cnt + mask[i]
            inv = jnp.float32(1.0)
            for n in (2, 3, 4, 5):
                inv = jnp.where(cnt > (n - 0.5), jnp.float32(1.0 / n), inv)
            for cp in copies:
                cp.wait()
            for c in range(_EMBED_DIM // _LANES):
                acc = rows_v[0, pl.ds(c * _LANES, _LANES)]
                for i in range(1, _NUM_SLOTS):
                    acc = acc + rows_v[i, pl.ds(c * _LANES, _LANES)]
                pooled_v[pl.ds(c * _LANES, _LANES)] = acc * inv
            pltpu.sync_copy(pooled_v, out_hbm)

    return _sc_pool


def _tc_mlp_body(feats_ref, w1_ref, b1_ref, w2_ref, b2_ref, w3_ref, b3_ref,
                 out_ref):
    h = jnp.dot(feats_ref[...], w1_ref[...],
                preferred_element_type=jnp.float32)
    h = jnp.maximum(h + b1_ref[...], 0.0)
    h = jnp.dot(h, w2_ref[...], preferred_element_type=jnp.float32)
    h = jnp.maximum(h + b2_ref[...], 0.0)
    out_ref[...] = (jnp.dot(h, w3_ref[...],
                            preferred_element_type=jnp.float32) + b3_ref[...])


def _tc_mlp(feats, W1, b1, W2, b2, W3, b3):
    return pl.pallas_call(
        _tc_mlp_body,
        out_shape=jax.ShapeDtypeStruct((1, W3.shape[1]), jnp.float32),
    )(feats, W1, b1, W2, b2, W3, b3)


def kernel(scalars, selection_mask, hand, hand_type, deck, jokers, emb_table,
           W1, b1, W2, b2, W3, b3):
    ids = jokers[:, 0].astype(jnp.int32)
    idx16 = jnp.zeros((_NUM_IDS,), jnp.int32).at[: ids.shape[0]].set(ids)
    pooled = _sc_pool_kernel()(emb_table, idx16)
    feats = jnp.concatenate([
        scalars, selection_mask, hand.reshape(-1), hand_type, deck, pooled,
        jokers[:, 1],
    ]).reshape(1, -1)
    out = _tc_mlp(feats, W1, b1.reshape(1, -1), W2, b2.reshape(1, -1), W3,
                  b3.reshape(1, -1))
    return out.reshape(-1)


# speedup vs baseline: 1.7456x; 1.0581x over previous
import jax
import jax.numpy as jnp
from jax.experimental import pallas as pl
from jax.experimental.pallas import tpu as pltpu

_EMBED_DIM = 32
_NUM_SLOTS = 5
_POOL_OFF = 216


def _fused_body(jok_smem, feats_ref, table_any, w1_ref, b1_ref, w2_ref,
                b2_ref, w3_ref, b3_ref, out_ref, rows_v, sem):
    ids = [jok_smem[i, 0].astype(jnp.int32) for i in range(_NUM_SLOTS)]
    copies = [
        pltpu.make_async_copy(
            table_any.at[pl.ds(ids[i], 1), :],
            rows_v.at[pl.ds(i, 1), :],
            sem,
        )
        for i in range(_NUM_SLOTS)
    ]
    for cp in copies:
        cp.start()
    cnt = jnp.float32(0.0)
    for i in range(_NUM_SLOTS):
        cnt = cnt + jnp.where(ids[i] > 0, jnp.float32(1.0), jnp.float32(0.0))
    inv = 1.0 / jnp.maximum(cnt, 1.0)
    for cp in copies:
        cp.wait()
    pooled = jnp.sum(rows_v[...], axis=0, keepdims=True) * inv
    h = jnp.dot(feats_ref[...], w1_ref[...],
                preferred_element_type=jnp.float32)
    h = h + jnp.dot(pooled, w1_ref[_POOL_OFF:_POOL_OFF + _EMBED_DIM, :],
                    preferred_element_type=jnp.float32)
    h = jnp.maximum(h + b1_ref[...], 0.0)
    h = jnp.dot(h, w2_ref[...], preferred_element_type=jnp.float32)
    h = jnp.maximum(h + b2_ref[...], 0.0)
    out_ref[...] = (jnp.dot(h, w3_ref[...],
                            preferred_element_type=jnp.float32) + b3_ref[...])


def kernel(scalars, selection_mask, hand, hand_type, deck, jokers, emb_table,
           W1, b1, W2, b2, W3, b3):
    feats_static = jnp.concatenate([
        scalars, selection_mask, hand.reshape(-1), hand_type, deck,
        jnp.zeros((_EMBED_DIM,), jnp.float32), jokers[:, 1],
    ]).reshape(1, -1)
    out = pl.pallas_call(
        _fused_body,
        in_specs=[
            pl.BlockSpec(memory_space=pltpu.MemorySpace.SMEM),
            pl.BlockSpec(memory_space=pltpu.MemorySpace.VMEM),
            pl.BlockSpec(memory_space=pltpu.MemorySpace.HBM),
            pl.BlockSpec(memory_space=pltpu.MemorySpace.VMEM),
            pl.BlockSpec(memory_space=pltpu.MemorySpace.VMEM),
            pl.BlockSpec(memory_space=pltpu.MemorySpace.VMEM),
            pl.BlockSpec(memory_space=pltpu.MemorySpace.VMEM),
            pl.BlockSpec(memory_space=pltpu.MemorySpace.VMEM),
            pl.BlockSpec(memory_space=pltpu.MemorySpace.VMEM),
        ],
        out_shape=jax.ShapeDtypeStruct((1, W3.shape[1]), jnp.float32),
        scratch_shapes=[
            pltpu.VMEM((_NUM_SLOTS, _EMBED_DIM), jnp.float32),
            pltpu.SemaphoreType.DMA,
        ],
    )(jokers, feats_static, emb_table, W1, b1.reshape(1, -1), W2,
      b2.reshape(1, -1), W3, b3.reshape(1, -1))
    return out.reshape(-1)


# speedup vs baseline: 88.4563x; 50.6740x over previous
import jax
import jax.numpy as jnp
from jax import lax
from jax.experimental import pallas as pl
from jax.experimental.pallas import tpu as pltpu

_EMBED_DIM = 32
_NUM_SLOTS = 5
_POOL_OFF = 216


def _fused_body(jokt_smem, feats_ref, tablet_hbm, w1_ref, b1_ref, w2_ref,
                b2_ref, w3t_ref, b3_ref, out_ref, blocks_v, sem):
    ids = [jokt_smem[0, i].astype(jnp.int32) for i in range(_NUM_SLOTS)]
    copies = [
        pltpu.make_async_copy(
            tablet_hbm.at[:, pl.ds(pl.multiple_of(
                (ids[i] // 128) * 128, 128), 128)],
            blocks_v.at[i],
            sem,
        )
        for i in range(_NUM_SLOTS)
    ]
    for cp in copies:
        cp.start()
    cnt = jnp.float32(0.0)
    for i in range(_NUM_SLOTS):
        cnt = cnt + jnp.where(ids[i] > 0, jnp.float32(1.0), jnp.float32(0.0))
    inv = 1.0 / jnp.maximum(cnt, 1.0)
    lane = lax.broadcasted_iota(jnp.int32, (1, 128), 1)
    for cp in copies:
        cp.wait()
    acc = jnp.zeros((_EMBED_DIM, 1), jnp.float32)
    for i in range(_NUM_SLOTS):
        onehot = (lane == (ids[i] % 128)).astype(jnp.float32)
        acc = acc + jnp.sum(blocks_v[i] * onehot, axis=1, keepdims=True)
    pooled_col = acc * inv
    h = jnp.dot(feats_ref[...], w1_ref[...],
                preferred_element_type=jnp.float32)
    h = h + lax.dot_general(
        pooled_col, w1_ref[_POOL_OFF:_POOL_OFF + _EMBED_DIM, :],
        (((0,), (0,)), ((), ())), preferred_element_type=jnp.float32)
    h = jnp.maximum(h + b1_ref[...], 0.0)
    h = jnp.dot(h, w2_ref[...], preferred_element_type=jnp.float32)
    h = jnp.maximum(h + b2_ref[...], 0.0)
    out_ref[...] = lax.dot_general(
        h, w3t_ref[...], (((1,), (1,)), ((), ())),
        preferred_element_type=jnp.float32) + b3_ref[...]


def kernel(scalars, selection_mask, hand, hand_type, deck, jokers, emb_table,
           W1, b1, W2, b2, W3, b3):
    feats_static = jnp.concatenate([
        scalars, selection_mask, hand.reshape(-1), hand_type, deck,
        jnp.zeros((_EMBED_DIM,), jnp.float32), jokers[:, 1],
    ]).reshape(1, -1)
    out = pl.pallas_call(
        _fused_body,
        in_specs=[
            pl.BlockSpec(memory_space=pltpu.MemorySpace.SMEM),
            pl.BlockSpec(memory_space=pltpu.MemorySpace.VMEM),
            pl.BlockSpec(memory_space=pltpu.MemorySpace.HBM),
            pl.BlockSpec(memory_space=pltpu.MemorySpace.VMEM),
            pl.BlockSpec(memory_space=pltpu.MemorySpace.VMEM),
            pl.BlockSpec(memory_space=pltpu.MemorySpace.VMEM),
            pl.BlockSpec(memory_space=pltpu.MemorySpace.VMEM),
            pl.BlockSpec(memory_space=pltpu.MemorySpace.VMEM),
            pl.BlockSpec(memory_space=pltpu.MemorySpace.VMEM),
        ],
        out_shape=jax.ShapeDtypeStruct((1, W3.shape[1]), jnp.float32),
        scratch_shapes=[
            pltpu.VMEM((_NUM_SLOTS, _EMBED_DIM, 128), jnp.float32),
            pltpu.SemaphoreType.DMA,
        ],
    )(jokers.T, feats_static, emb_table.T, W1, b1.reshape(1, -1), W2,
      b2.reshape(1, -1), W3.T, b3.reshape(1, -1))
    return out.reshape(-1)


# speedup vs baseline: 172.7392x; 1.9528x over previous
import jax
import jax.numpy as jnp
from jax import lax
from jax.experimental import pallas as pl
from jax.experimental.pallas import tpu as pltpu

_EMBED_DIM = 32
_NUM_SLOTS = 5
_POOL_OFF = 216


def _dot(a, b):
    return jnp.dot(a, b, preferred_element_type=jnp.float32)


def _fused_body(jokt_smem, scalars_ref, sel_ref, hand_ref, ht_ref, deck_ref,
                tablet_hbm, w1_ref, b1_ref, w2_ref, b2_ref, w3t_ref, b3_ref,
                out_ref, blocks_v, sem):
    ids = [jokt_smem[0, i].astype(jnp.int32) for i in range(_NUM_SLOTS)]
    copies = [
        pltpu.make_async_copy(
            tablet_hbm.at[:, pl.ds(pl.multiple_of(
                (ids[i] // 128) * 128, 128), 128)],
            blocks_v.at[i],
            sem,
        )
        for i in range(_NUM_SLOTS)
    ]
    for cp in copies:
        cp.start()

    h = _dot(scalars_ref[...], w1_ref[0:16, :])
    h = h + _dot(sel_ref[...], w1_ref[16:24, :])
    for r in range(8):
        h = h + _dot(hand_ref[pl.ds(r, 1), :], w1_ref[24 + 16 * r:40 + 16 * r, :])
    htdeck = jnp.concatenate([ht_ref[...], deck_ref[...]], axis=1)
    h = h + _dot(htdeck, w1_ref[152:216, :])
    lane8 = lax.broadcasted_iota(jnp.int32, (1, 8), 1)
    enabled = jnp.zeros((1, 8), jnp.float32)
    for i in range(_NUM_SLOTS):
        enabled = enabled + jnp.where(lane8 == i, jokt_smem[1, i],
                                      jnp.float32(0.0))
    h = h + _dot(enabled[:, 0:_NUM_SLOTS], w1_ref[248:253, :])

    cnt = jnp.float32(0.0)
    for i in range(_NUM_SLOTS):
        cnt = cnt + jnp.where(ids[i] > 0, jnp.float32(1.0), jnp.float32(0.0))
    inv = 1.0 / jnp.maximum(cnt, 1.0)
    lane = lax.broadcasted_iota(jnp.int32, (1, 128), 1)
    for cp in copies:
        cp.wait()
    acc = jnp.zeros((_EMBED_DIM, 1), jnp.float32)
    for i in range(_NUM_SLOTS):
        onehot = (lane == (ids[i] % 128)).astype(jnp.float32)
        acc = acc + jnp.sum(blocks_v[i] * onehot, axis=1, keepdims=True)
    pooled_col = acc * inv
    h = h + lax.dot_general(
        pooled_col, w1_ref[_POOL_OFF:_POOL_OFF + _EMBED_DIM, :],
        (((0,), (0,)), ((), ())), preferred_element_type=jnp.float32)

    h = jnp.maximum(h + b1_ref[...], 0.0)
    h = jnp.maximum(_dot(h, w2_ref[...]) + b2_ref[...], 0.0)
    out_ref[...] = lax.dot_general(
        h, w3t_ref[...], (((1,), (1,)), ((), ())),
        preferred_element_type=jnp.float32) + b3_ref[...]


def kernel(scalars, selection_mask, hand, hand_type, deck, jokers, emb_table,
           W1, b1, W2, b2, W3, b3):
    vmem = pl.BlockSpec(memory_space=pltpu.MemorySpace.VMEM)
    out = pl.pallas_call(
        _fused_body,
        in_specs=[
            pl.BlockSpec(memory_space=pltpu.MemorySpace.SMEM),
            vmem, vmem, vmem, vmem, vmem,
            pl.BlockSpec(memory_space=pltpu.MemorySpace.HBM),
            vmem, vmem, vmem, vmem, vmem, vmem,
        ],
        out_shape=jax.ShapeDtypeStruct((1, W3.shape[1]), jnp.float32),
        scratch_shapes=[
            pltpu.VMEM((_NUM_SLOTS, _EMBED_DIM, 128), jnp.float32),
            pltpu.SemaphoreType.DMA,
        ],
    )(jokers.T, scalars.reshape(1, -1), selection_mask.reshape(1, -1), hand,
      hand_type.reshape(1, -1), deck.reshape(1, -1), emb_table.T, W1,
      b1.reshape(1, -1), W2, b2.reshape(1, -1), W3.T, b3.reshape(1, -1))
    return out.reshape(-1)
